# Initial kernel scaffold; baseline (speedup 1.0000x reference)
import jax
import jax.numpy as jnp
from jax.experimental import pallas as pl


def kernel(x, pos_table):
    raise NotImplementedError("write your pallas kernel here")


# speedup vs baseline: 3.3992x; 3.3992x over previous
import jax
import jax.numpy as jnp
from jax.experimental import pallas as pl

BS = 1024


def _add_body(x_ref, pos_ref, o_ref):
    o_ref[...] = x_ref[...] + pos_ref[...]


def kernel(x, pos_table):
    B, S, E = x.shape
    grid = (S // BS, B)
    return pl.pallas_call(
        _add_body,
        grid=grid,
        in_specs=[
            pl.BlockSpec((1, BS, E), lambda si, b: (b, si, 0)),
            pl.BlockSpec((BS, E), lambda si, b: (si, 0)),
        ],
        out_specs=pl.BlockSpec((1, BS, E), lambda si, b: (b, si, 0)),
        out_shape=jax.ShapeDtypeStruct((B, S, E), x.dtype),
    )(x, pos_table)


# speedup vs baseline: 3.6179x; 1.0644x over previous
import jax
import jax.numpy as jnp
from jax.experimental import pallas as pl

BS = 2048


def _add_body(x_ref, pos_ref, o_ref):
    o_ref[...] = x_ref[...] + pos_ref[...]


def kernel(x, pos_table):
    B, S, E = x.shape
    grid = (S // BS, B)
    return pl.pallas_call(
        _add_body,
        grid=grid,
        in_specs=[
            pl.BlockSpec((1, BS, E), lambda si, b: (b, si, 0)),
            pl.BlockSpec((BS, E), lambda si, b: (si, 0)),
        ],
        out_specs=pl.BlockSpec((1, BS, E), lambda si, b: (b, si, 0)),
        out_shape=jax.ShapeDtypeStruct((B, S, E), x.dtype),
    )(x, pos_table)


# speedup vs baseline: 3.6423x; 1.0067x over previous
import jax
import jax.numpy as jnp
from jax.experimental import pallas as pl

BS = 3072


def _add_body(x_ref, pos_ref, o_ref):
    o_ref[...] = x_ref[...] + pos_ref[...]


def kernel(x, pos_table):
    B, S, E = x.shape
    grid = (pl.cdiv(S, BS), B)
    return pl.pallas_call(
        _add_body,
        grid=grid,
        in_specs=[
            pl.BlockSpec((1, BS, E), lambda si, b: (b, si, 0)),
            pl.BlockSpec((BS, E), lambda si, b: (si, 0)),
        ],
        out_specs=pl.BlockSpec((1, BS, E), lambda si, b: (b, si, 0)),
        out_shape=jax.ShapeDtypeStruct((B, S, E), x.dtype),
    )(x, pos_table)
